# Initial kernel scaffold; baseline (speedup 1.0000x reference)
import functools

import jax
import jax.numpy as jnp
from jax import lax
from jax.experimental import pallas as pl
from jax.experimental.pallas import tpu as pltpu
from jax.experimental.pallas import tpu_sc as plsc

N = 10000
E = 320000
D = 128
H = 128

NC = 2
NS = 16
NW = NC * NS
C = 128
CHUNKS = E // C
K_STEPS = (CHUNKS + NW - 1) // NW
ROWS_PER_SUB = N // NS
ZROWS = 125

_SLOPE = (0.125 + 1.0 / 3.0) / 2.0


def _sc_body(x_hbm, ei_hbm, eh_hbm, norm_hbm, out_hbm,
             table, ebuf, nbuf, ibuf, dbuf, zbuf, sem):
    cid = lax.axis_index("c")
    sid = lax.axis_index("s")
    wid = sid * NC + cid

    def zrow(r, carry):
        for j in range(D // 16):
            zbuf[r, pl.ds(j * 16, 16)] = jnp.zeros((16,), jnp.float32)
        return carry
    lax.fori_loop(0, ZROWS, zrow, 0)

    for term in range(2):
        for k in range(ROWS_PER_SUB // ZROWS):
            pltpu.sync_copy(zbuf, table.at[pl.ds(sid * ROWS_PER_SUB + k * ZROWS, ZROWS)])
        plsc.subcore_barrier()

        def chunk_body(k, carry):
            g = wid + NW * k

            @pl.when(g < CHUNKS)
            def _():
                base = g * C
                pltpu.sync_copy(ei_hbm.at[1, pl.ds(base, C)], dbuf)
                pltpu.sync_copy(norm_hbm.at[pl.ds(base, C)], nbuf)
                if term == 0:
                    pltpu.sync_copy(ei_hbm.at[0, pl.ds(base, C)], ibuf)
                    pltpu.async_copy(x_hbm.at[ibuf], ebuf, sem).wait()
                else:
                    pltpu.sync_copy(eh_hbm.at[pl.ds(base, C)], ebuf)

                def row_body(e, carry2):
                    sv = jnp.full((16,), nbuf[e, 0], jnp.float32)
                    for j in range(D // 16):
                        ebuf[e, pl.ds(j * 16, 16)] = ebuf[e, pl.ds(j * 16, 16)] * sv
                    return carry2
                lax.fori_loop(0, C, row_body, 0)

                pltpu.sync_copy(ebuf, table.at[dbuf], add=True)
            return carry
        lax.fori_loop(0, K_STEPS, chunk_body, 0)
        plsc.subcore_barrier()

        pltpu.sync_copy(table.at[pl.ds(sid * ROWS_PER_SUB, ROWS_PER_SUB)],
                        out_hbm.at[term, cid, pl.ds(sid * ROWS_PER_SUB, ROWS_PER_SUB)])
        plsc.subcore_barrier()


_sc_scatter = functools.partial(
    pl.kernel,
    out_type=jax.ShapeDtypeStruct((2, NC, N, D), jnp.float32),
    mesh=plsc.VectorSubcoreMesh(core_axis_name="c", subcore_axis_name="s"),
    scratch_types=[
        pltpu.VMEM_SHARED((N, D), jnp.float32),
        pltpu.VMEM((C, D), jnp.float32),
        pltpu.VMEM((C, 1), jnp.float32),
        pltpu.VMEM((C,), jnp.int32),
        pltpu.VMEM((C,), jnp.int32),
        pltpu.VMEM((ZROWS, D), jnp.float32),
        pltpu.SemaphoreType.DMA,
    ],
)(_sc_body)


BLK = 1000


def _tc_body(p_ref, w_ref, o_ref):
    a = p_ref[0, 0] + p_ref[0, 1]
    b = p_ref[1, 0] + p_ref[1, 1]
    w = w_ref[...]
    acc = lax.dot_general(a, w[:, :D], (((1,), (1,)), ((), ())),
                          preferred_element_type=jnp.float32)
    acc = acc + lax.dot_general(b, w[:, D:], (((1,), (1,)), ((), ())),
                                preferred_element_type=jnp.float32)
    o_ref[...] = jnp.where(acc >= 0, acc, acc * _SLOPE)


def _tc_finish(parts, W):
    return pl.pallas_call(
        _tc_body,
        out_shape=jax.ShapeDtypeStruct((N, H), jnp.float32),
        grid=(N // BLK,),
        in_specs=[
            pl.BlockSpec((2, NC, BLK, D), lambda i: (0, 0, i, 0)),
            pl.BlockSpec((H, 2 * D), lambda i: (0, 0)),
        ],
        out_specs=pl.BlockSpec((BLK, H), lambda i: (i, 0)),
    )(parts, W)


def kernel(x, edge_index, edge_h, norm, W):
    parts = _sc_scatter(x, edge_index, edge_h, norm)
    return _tc_finish(parts, W)


# speedup vs baseline: 3.1180x; 3.1180x over previous
import functools

import jax
import jax.numpy as jnp
from jax import lax
from jax.experimental import pallas as pl
from jax.experimental.pallas import tpu as pltpu
from jax.experimental.pallas import tpu_sc as plsc

N = 10000
E = 320000
D = 128
H = 128

NC = 2
NS = 16
NW = NC * NS
C = 128
CHUNKS = E // C
K_STEPS = (CHUNKS + NW - 1) // NW
RPS = 624
ZROWS = 208

_SLOPE = (0.125 + 1.0 / 3.0) / 2.0


def _sc_body(x_hbm, src_hbm, dst_hbm, eh_hbm, norm_hbm, out_hbm,
             table, ebuf, nbuf, ibuf, dbuf, zbuf, sem):
    cid = lax.axis_index("c")
    sid = lax.axis_index("s")
    wid = sid * NC + cid

    def zrow(r, carry):
        for j in range(D // 16):
            zbuf[r, pl.ds(j * 16, 16)] = jnp.zeros((16,), jnp.float32)
        return carry
    lax.fori_loop(0, ZROWS, zrow, 0)

    for term in range(2):
        for k in range(RPS // ZROWS):
            pltpu.sync_copy(zbuf, table.at[pl.ds(sid * RPS + k * ZROWS, ZROWS)])

        @pl.when(sid == NS - 1)
        def _():
            pltpu.sync_copy(zbuf.at[pl.ds(0, 16)], table.at[pl.ds(NS * RPS, N - NS * RPS)])
        plsc.subcore_barrier()

        def chunk_body(k, carry):
            g = wid + NW * k

            @pl.when(g < CHUNKS)
            def _():
                base = g * C
                pltpu.sync_copy(dst_hbm.at[pl.ds(base, C)], dbuf)
                pltpu.sync_copy(norm_hbm.at[pl.ds(base, C)], nbuf)
                if term == 0:
                    pltpu.sync_copy(src_hbm.at[pl.ds(base, C)], ibuf)
                    pltpu.async_copy(x_hbm.at[ibuf], ebuf, sem).wait()
                else:
                    pltpu.sync_copy(eh_hbm.at[pl.ds(base, C)], ebuf)

                def grp_body(gi, carry2):
                    nv = nbuf[pl.ds(gi * 16, 16)]
                    for l in range(16):
                        sv = jnp.full((16,), nv[l], jnp.float32)
                        e = gi * 16 + l
                        for j in range(D // 16):
                            ebuf[e, pl.ds(j * 16, 16)] = ebuf[e, pl.ds(j * 16, 16)] * sv
                    return carry2
                lax.fori_loop(0, C // 16, grp_body, 0)

                pltpu.sync_copy(ebuf, table.at[dbuf], add=True)
            return carry
        lax.fori_loop(0, K_STEPS, chunk_body, 0)
        plsc.subcore_barrier()

        pltpu.sync_copy(table.at[pl.ds(sid * RPS, RPS)],
                        out_hbm.at[term, cid, pl.ds(sid * RPS, RPS)])

        @pl.when(sid == NS - 1)
        def _():
            pltpu.sync_copy(table.at[pl.ds(NS * RPS, N - NS * RPS)],
                            out_hbm.at[term, cid, pl.ds(NS * RPS, N - NS * RPS)])
        plsc.subcore_barrier()


_sc_scatter = functools.partial(
    pl.kernel,
    out_type=jax.ShapeDtypeStruct((2, NC, N, D), jnp.float32),
    mesh=plsc.VectorSubcoreMesh(core_axis_name="c", subcore_axis_name="s"),
    scratch_types=[
        pltpu.VMEM_SHARED((N, D), jnp.float32),
        pltpu.VMEM((C, D), jnp.float32),
        pltpu.VMEM((C,), jnp.float32),
        pltpu.VMEM((C,), jnp.int32),
        pltpu.VMEM((C,), jnp.int32),
        pltpu.VMEM((ZROWS, D), jnp.float32),
        pltpu.SemaphoreType.DMA,
    ],
)(_sc_body)


BLK = 1000


def _tc_body(p_ref, w_ref, o_ref):
    a = p_ref[0, 0] + p_ref[0, 1]
    b = p_ref[1, 0] + p_ref[1, 1]
    w = w_ref[...]
    acc = lax.dot_general(a, w[:, :D], (((1,), (1,)), ((), ())),
                          preferred_element_type=jnp.float32)
    acc = acc + lax.dot_general(b, w[:, D:], (((1,), (1,)), ((), ())),
                                preferred_element_type=jnp.float32)
    o_ref[...] = jnp.where(acc >= 0, acc, acc * _SLOPE)


def _tc_finish(parts, W):
    return pl.pallas_call(
        _tc_body,
        out_shape=jax.ShapeDtypeStruct((N, H), jnp.float32),
        grid=(N // BLK,),
        in_specs=[
            pl.BlockSpec((2, NC, BLK, D), lambda i: (0, 0, i, 0)),
            pl.BlockSpec((H, 2 * D), lambda i: (0, 0)),
        ],
        out_specs=pl.BlockSpec((BLK, H), lambda i: (i, 0)),
    )(parts, W)


def kernel(x, edge_index, edge_h, norm, W):
    parts = _sc_scatter(x, edge_index[0], edge_index[1], edge_h, norm.reshape(E))
    return _tc_finish(parts, W)
